# baseline (speedup 1.0000x reference)
import functools

import jax
import jax.numpy as jnp
from jax import lax
from jax.experimental import pallas as pl
from jax.experimental.pallas import tpu as pltpu
from jax.experimental.pallas import tpu_sc as plsc

N_NODES = 100000
N_ELEMS = 100000
NPAD = 100352
EPAD = 100352
KEY_F = 131072
NW = 32
SLICE = NPAD // NW
CHUNKS = SLICE // 16
FILLB = 784
ROUNDS = 7
CH = EPAD // ROUNDS
RED = CH // 16
NB = 448
NBCH = NB // 16
SLOTS = 16

_mesh = plsc.VectorSubcoreMesh(core_axis_name="c", subcore_axis_name="s")
_sc_params = pltpu.CompilerParams(needs_layout_passes=False)


@functools.partial(
    pl.kernel,
    out_type=(
        jax.ShapeDtypeStruct((2 * EPAD,), jnp.int32),
        jax.ShapeDtypeStruct((2 * EPAD,), jnp.int32),
    ),
    mesh=_mesh,
    compiler_params=_sc_params,
    scratch_types=[
        pltpu.VMEM((EPAD,), jnp.int32),
        pltpu.VMEM((FILLB * 4,), jnp.int32),
        pltpu.VMEM((FILLB * 4,), jnp.int32),
        pltpu.VMEM((FILLB * 4,), jnp.float32),
        pltpu.VMEM_SHARED((16 * CH,), jnp.int32),
        pltpu.VMEM((RED,), jnp.int32),
        pltpu.VMEM((RED,), jnp.int32),
    ],
)
def _s1(eid_hbm, isa_hbm, mask_hbm, mka_hbm, mkb_hbm,
        tab, eid_b, isa_b, mask_b, spm, rbuf, racc):
    c = lax.axis_index("c")
    s = lax.axis_index("s")
    node_base = (c * 16 + s) * SLICE
    iota = lax.iota(jnp.int32, 16)
    neg1 = jnp.full((16,), -1, jnp.int32)

    for out_ref, want in ((mka_hbm, 1), (mkb_hbm, 0)):
        def init_body(j, _):
            tab[pl.ds(j * 64, 16)] = neg1
            tab[pl.ds(j * 64 + 16, 16)] = neg1
            tab[pl.ds(j * 64 + 32, 16)] = neg1
            tab[pl.ds(j * 64 + 48, 16)] = neg1
            return 0
        lax.fori_loop(0, EPAD // 64, init_body, 0)

        for b in range(SLICE // FILLB):
            base = node_base + b * FILLB
            pltpu.sync_copy(eid_hbm.at[pl.ds(base * 4, FILLB * 4)], eid_b)
            pltpu.sync_copy(isa_hbm.at[pl.ds(base * 4, FILLB * 4)], isa_b)
            pltpu.sync_copy(mask_hbm.at[pl.ds(base * 4, FILLB * 4)], mask_b)

            def fill_body(k, _):
                sl = pl.ds(k * 16, 16)
                g = k * 16 + iota
                eidv = eid_b[sl]
                valid = (mask_b[sl] > 0.5) & (isa_b[sl] == want)
                key = (g & 3) * KEY_F + (base + (g >> 2))
                cur = plsc.load_gather(tab, [eidv])
                plsc.store_scatter(tab, [eidv], jnp.maximum(cur, key),
                                   mask=valid)
                return 0
            lax.fori_loop(0, FILLB * 4 // 16, fill_body, 0)

        for r in range(ROUNDS):
            pltpu.sync_copy(tab.at[pl.ds(r * CH, CH)],
                            spm.at[pl.ds(s * CH, CH)])
            plsc.subcore_barrier()
            myoff = s * RED
            pltpu.sync_copy(spm.at[pl.ds(myoff, RED)], racc)
            for t in range(1, 16):
                pltpu.sync_copy(spm.at[pl.ds(t * CH + myoff, RED)], rbuf)

                def red_body(j, _):
                    sl = pl.ds(j * 16, 16)
                    racc[sl] = jnp.maximum(racc[sl], rbuf[sl])
                    return 0
                lax.fori_loop(0, RED // 16, red_body, 0)
            pltpu.sync_copy(
                racc, out_ref.at[pl.ds(c * EPAD + r * CH + myoff, RED)])
            plsc.subcore_barrier()


@functools.partial(
    pl.kernel,
    out_type=jax.ShapeDtypeStruct((2 * 16 * 256,), jnp.float32),
    mesh=_mesh,
    compiler_params=_sc_params,
    scratch_types=(
        [pltpu.VMEM((SLICE,), jnp.int32) for _ in range(2)]
        + [pltpu.VMEM((SLICE,), jnp.int32)]
        + [pltpu.VMEM((2 * SLICE,), jnp.int32)]
        + [pltpu.VMEM((3 * SLICE,), jnp.float32)]
        + [pltpu.VMEM((SLICE,), jnp.float32) for _ in range(4)]
        + [pltpu.VMEM((SLICE,), jnp.int32) for _ in range(12)]
        + [pltpu.VMEM((SLICE,), jnp.float32) for _ in range(12)]
        + [pltpu.VMEM((NB * 15,), jnp.float32),
           pltpu.VMEM((NB * 4,), jnp.float32),
           pltpu.VMEM((NB,), jnp.float32),
           pltpu.VMEM((NB,), jnp.float32),
           pltpu.VMEM((NB * 3,), jnp.float32)]
        + [pltpu.VMEM((256,), jnp.float32),
           pltpu.VMEM_SHARED((16 * 256,), jnp.float32),
           pltpu.SemaphoreType.DMA]
    ),
)
def _s2(mka_hbm, mkb_hbm, pred_hbm, conn_hbm, dirs_hbm, len_hbm,
        pe_hbm, pa_hbm, pi_hbm, fm_hbm, bcd_hbm, bcr_hbm, fe_hbm,
        part_hbm,
        mka, mkb, tmp, conb, dirb, lb, peb, pab, pib,
        ixa0, ixa1, ixa2, ixb0, ixb1, ixb2,
        ixda0, ixda1, ixda2, ixdb0, ixdb1, ixdb2,
        ga0, ga1, ga2, gb0, gb1, gb2,
        gda0, gda1, gda2, gdb0, gdb1, gdb2,
        pnb, fmb, bcdb, bcrb, feb,
        obuf, spmf, sem):
    c = lax.axis_index("c")
    s = lax.axis_index("s")
    base = (c * 16 + s) * SLICE
    iota = lax.iota(jnp.int32, 16)

    pltpu.sync_copy(mka_hbm.at[pl.ds(base, SLICE)], mka)
    pltpu.sync_copy(mka_hbm.at[pl.ds(EPAD + base, SLICE)], tmp)

    def fold_body(k, ref_pair):
        sl = pl.ds(k * 16, 16)
        mka[sl] = jnp.maximum(mka[sl], tmp[sl])
        return ref_pair
    lax.fori_loop(0, CHUNKS, fold_body, 0)
    pltpu.sync_copy(mkb_hbm.at[pl.ds(base, SLICE)], mkb)
    pltpu.sync_copy(mkb_hbm.at[pl.ds(EPAD + base, SLICE)], tmp)

    def fold_body_b(k, x):
        sl = pl.ds(k * 16, 16)
        mkb[sl] = jnp.maximum(mkb[sl], tmp[sl])
        return x
    lax.fori_loop(0, CHUNKS, fold_body_b, 0)

    pltpu.sync_copy(conn_hbm.at[pl.ds(2 * base, 2 * SLICE)], conb)
    pltpu.sync_copy(dirs_hbm.at[pl.ds(3 * base, 3 * SLICE)], dirb)
    pltpu.sync_copy(len_hbm.at[pl.ds(base, SLICE)], lb)
    pltpu.sync_copy(pe_hbm.at[pl.ds(base, SLICE)], peb)
    pltpu.sync_copy(pa_hbm.at[pl.ds(base, SLICE)], pab)
    pltpu.sync_copy(pi_hbm.at[pl.ds(base, SLICE)], pib)

    ixa = (ixa0, ixa1, ixa2)
    ixb = (ixb0, ixb1, ixb2)
    ixda = (ixda0, ixda1, ixda2)
    ixdb = (ixdb0, ixdb1, ixdb2)

    def idx_body(k, _):
        sl = pl.ds(k * 16, 16)
        gid = base + k * 16 + iota
        spread = gid * 14
        a = mka[sl]
        b = mkb[sl]
        rowa = 15 * (a & (KEY_F - 1)) + 3 * (a >> 17) + 3
        rowb = 15 * (b & (KEY_F - 1)) + 3 * (b >> 17) + 3
        e2 = (k * 16 + iota) * 2
        e3 = (k * 16 + iota) * 3
        cna = plsc.load_gather(conb, [e2])
        cnb_ = plsc.load_gather(conb, [e2 + 1])
        for comp in range(3):
            ixa[comp][sl] = jnp.where(a >= 0, rowa + comp, spread)
            ixb[comp][sl] = jnp.where(b >= 0, rowb + comp, spread)
            ixda[comp][sl] = 15 * cna + comp
            ixdb[comp][sl] = 15 * cnb_ + comp
        return 0
    lax.fori_loop(0, CHUNKS, idx_body, 0)

    copies = []
    for ix, dst in ((ixa0, ga0), (ixa1, ga1), (ixa2, ga2),
                    (ixb0, gb0), (ixb1, gb1), (ixb2, gb2),
                    (ixda0, gda0), (ixda1, gda1), (ixda2, gda2),
                    (ixdb0, gdb0), (ixdb1, gdb1), (ixdb2, gdb2)):
        copies.append(pltpu.async_copy(pred_hbm.at[ix], dst, sem))

    zero = jnp.zeros((16,), jnp.float32)
    nacc = [zero] * 11
    for blk in range(SLICE // NB):
        nb0 = base + blk * NB
        pltpu.sync_copy(pred_hbm.at[pl.ds(nb0 * 15, NB * 15)], pnb)
        pltpu.sync_copy(fm_hbm.at[pl.ds(nb0 * 4, NB * 4)], fmb)
        pltpu.sync_copy(bcd_hbm.at[pl.ds(nb0, NB)], bcdb)
        pltpu.sync_copy(bcr_hbm.at[pl.ds(nb0, NB)], bcrb)
        pltpu.sync_copy(fe_hbm.at[pl.ds(nb0 * 3, NB * 3)], feb)

        def node_body(k, carry):
            (aq0, aq1, acf, aq2, aq3, acff, aq4, aq5, aq6, acsd, acsr) = carry
            sl = pl.ds(k * 16, 16)
            lane = k * 16 + iota
            m = ((nb0 + lane) < N_NODES).astype(jnp.float32)
            p15 = lane * 15
            cols = [plsc.load_gather(pnb, [p15 + cc]) for cc in range(15)]
            bcd = bcdb[sl]
            bcr = bcrb[sl]
            free = m * (bcd < 0.5).astype(jnp.float32)
            supd = m * (bcd > 0.5).astype(jnp.float32)
            supr = m * (bcr > 0.5).astype(jnp.float32)
            f3 = lane * 3
            r0 = cols[3] + cols[6] + cols[9] + cols[12] \
                - plsc.load_gather(feb, [f3])
            r1 = cols[4] + cols[7] + cols[10] + cols[13] \
                - plsc.load_gather(feb, [f3 + 1])
            r2 = cols[5] + cols[8] + cols[11] + cols[14] \
                - plsc.load_gather(feb, [f3 + 2])
            aq0 = aq0 + free * (r0 * r0 + r1 * r1)
            aq1 = aq1 + free * (r2 * r2)
            acf = acf + free
            f4 = lane * 4
            for f in range(4):
                ffree = m * (plsc.load_gather(fmb, [f4 + f]) < 0.5).astype(
                    jnp.float32)
                g0 = cols[3 + 3 * f]
                g1 = cols[4 + 3 * f]
                g2 = cols[5 + 3 * f]
                aq2 = aq2 + ffree * (g0 * g0 + g1 * g1)
                aq3 = aq3 + ffree * (g2 * g2)
                acff = acff + ffree
            aq4 = aq4 + supd * cols[0] * cols[0]
            aq5 = aq5 + supd * cols[1] * cols[1]
            aq6 = aq6 + supr * cols[2] * cols[2]
            acsd = acsd + supd
            acsr = acsr + supr
            return (aq0, aq1, acf, aq2, aq3, acff, aq4, aq5, aq6, acsd, acsr)

        nacc = list(lax.fori_loop(0, NBCH, node_body, tuple(nacc)))

    for cp in copies:
        cp.wait()

    def phys_body(k, carry):
        acc_n, acc_m, acc_v = carry
        sl = pl.ds(k * 16, 16)
        e3 = (k * 16 + iota) * 3
        oka = (mka[sl] >= 0).astype(jnp.float32)
        okb = (mkb[sl] >= 0).astype(jnp.float32)

        fa0 = ga0[sl] * oka
        fa1 = ga1[sl] * oka
        fa2 = ga2[sl] * oka
        fb0 = gb0[sl] * okb
        fb1 = gb1[sl] * okb
        fb2 = gb2[sl] * okb
        da0 = gda0[sl]
        da1 = gda1[sl]
        da2 = gda2[sl]
        db0 = gdb0[sl]
        db1 = gdb1[sl]
        db2 = gdb2[sl]

        cs = plsc.load_gather(dirb, [e3])
        sn = plsc.load_gather(dirb, [e3 + 2])
        lv = lb[sl]
        ea = peb[sl] * pab[sl]
        ei = peb[sl] * pib[sl]

        u_a = da0 * cs + da1 * sn
        w_a = -da0 * sn + da1 * cs
        t_a = da2
        u_b = db0 * cs + db1 * sn
        w_b = -db0 * sn + db1 * cs
        t_b = db2
        fra0 = fa0 * cs + fa1 * sn
        fra1 = -fa0 * sn + fa1 * cs
        fra2 = fa2
        frb0 = fb0 * cs + fb1 * sn
        frb1 = -fb0 * sn + fb1 * cs
        frb2 = fb2

        l2 = lv * lv
        l3 = l2 * lv
        n_sf = ea * (u_b - u_a) / lv
        m_a = ei / l2 * (-6.0 * w_a - 4.0 * lv * t_a + 6.0 * w_b - 2.0 * lv * t_b)
        m_b = ei / l2 * (6.0 * w_a + 2.0 * lv * t_a - 6.0 * w_b + 4.0 * lv * t_b)
        v_sf = ei / l3 * (12.0 * w_a + 6.0 * lv * t_a - 12.0 * w_b + 6.0 * lv * t_b)

        m = ((base + k * 16 + iota) < N_ELEMS).astype(jnp.float32)
        rn0 = fra0 + n_sf
        rn1 = frb0 - n_sf
        rm0 = fra2 + m_a
        rm1 = frb2 - m_b
        rv0 = fra1 + v_sf
        rv1 = frb1 - v_sf
        acc_n = acc_n + m * (rn0 * rn0 + rn1 * rn1)
        acc_m = acc_m + m * (rm0 * rm0 + rm1 * rm1)
        acc_v = acc_v + m * (rv0 * rv0 + rv1 * rv1)
        return acc_n, acc_m, acc_v

    acc_n, acc_m, acc_v = lax.fori_loop(0, CHUNKS, phys_body,
                                        (zero, zero, zero))

    slots = [acc_n, acc_m, acc_v] + nacc
    for i, v in enumerate(slots):
        obuf[pl.ds(i * SLOTS, SLOTS)] = v
    obuf[pl.ds(14 * SLOTS, SLOTS)] = zero
    obuf[pl.ds(15 * SLOTS, SLOTS)] = zero
    pltpu.sync_copy(obuf, spmf.at[pl.ds(s * 256, 256)])
    plsc.subcore_barrier()

    @pl.when(s == 0)
    def _():
        pltpu.sync_copy(spmf, part_hbm.at[pl.ds(c * 16 * 256, 16 * 256)])


def _final_tc_body(ep_ref, fc_ref, mc_ref, uc_ref, tc_ref, out_ref):
    ep = ep_ref[...]
    fc = fc_ref[0, 0]
    mc = mc_ref[0, 0]
    uc = uc_ref[0, 0]
    th = tc_ref[0, 0]
    fc2 = fc * fc
    mc2 = mc * mc

    def slot(i):
        return jnp.sum(ep[:, 16 * i:16 * (i + 1)])

    l_eq = (slot(3) / fc2 + slot(4) / mc2) / jnp.maximum(slot(5), 1.0)
    l_free = (slot(6) / fc2 + slot(7) / mc2) / jnp.maximum(slot(8) * 3.0, 1.0)
    l_sup = ((slot(9) + slot(10)) / (uc * uc) / jnp.maximum(slot(12), 1.0)
             + slot(11) / (th * th) / jnp.maximum(slot(13), 1.0))
    e_cnt = float(N_ELEMS)
    total = (l_eq + l_free + l_sup
             + slot(0) / fc2 / e_cnt + slot(1) / mc2 / e_cnt
             + slot(2) / fc2 / e_cnt)
    out_ref[...] = jnp.reshape(total, (1, 1))


def _final_tc(ep, fc, mc, uc, th):
    return pl.pallas_call(
        _final_tc_body,
        in_specs=[
            pl.BlockSpec((NW, 256), lambda: (0, 0)),
            pl.BlockSpec((1, 1), lambda: (0, 0)),
            pl.BlockSpec((1, 1), lambda: (0, 0)),
            pl.BlockSpec((1, 1), lambda: (0, 0)),
            pl.BlockSpec((1, 1), lambda: (0, 0)),
        ],
        out_specs=pl.BlockSpec((1, 1), lambda: (0, 0)),
        out_shape=jax.ShapeDtypeStruct((1, 1), jnp.float32),
    )(ep, fc, mc, uc, th)


def kernel(pred, connectivity, face_element_id, face_is_A_end, face_mask,
           F_ext, bc_disp, bc_rot, elem_directions, elem_lengths,
           prop_E, prop_A, prop_I22, F_c, M_c, u_c, theta_c):
    n = pred.shape[0]
    e = connectivity.shape[0]
    pn = NPAD - n
    pe = EPAD - e

    eid_flat = jnp.pad(face_element_id, ((0, pn), (0, 0))).reshape(-1)
    isa_flat = jnp.pad(face_is_A_end, ((0, pn), (0, 0))).reshape(-1)
    mask_flat = jnp.pad(face_mask, ((0, pn), (0, 0)),
                        constant_values=0.5).reshape(-1)
    pred_flat = jnp.pad(pred, ((0, pn), (0, 0))).reshape(-1)
    conn_flat = jnp.pad(connectivity, ((0, pe), (0, 0))).reshape(-1)
    dirs_flat = jnp.pad(elem_directions, ((0, pe), (0, 0))).reshape(-1)
    len_p = jnp.pad(elem_lengths, (0, pe), constant_values=1.0)
    pe_p = jnp.pad(prop_E, (0, pe), constant_values=1.0)
    pa_p = jnp.pad(prop_A, (0, pe), constant_values=1.0)
    pi_p = jnp.pad(prop_I22, (0, pe), constant_values=1.0)
    bcd_flat = jnp.pad(bc_disp, ((0, pn), (0, 0)),
                       constant_values=0.5).reshape(-1)
    bcr_flat = jnp.pad(bc_rot, ((0, pn), (0, 0)),
                       constant_values=0.5).reshape(-1)
    fe_flat = jnp.pad(F_ext, ((0, pn), (0, 0))).reshape(-1)

    mka, mkb = _s1(eid_flat, isa_flat, mask_flat)
    part = _s2(mka, mkb, pred_flat, conn_flat, dirs_flat,
               len_p, pe_p, pa_p, pi_p, mask_flat, bcd_flat, bcr_flat,
               fe_flat)
    out = _final_tc(part.reshape(NW, 256), F_c.reshape(1, 1),
                    M_c.reshape(1, 1), u_c.reshape(1, 1),
                    theta_c.reshape(1, 1))
    return out[0, 0]


# speedup vs baseline: 1.4466x; 1.4466x over previous
import functools

import jax
import jax.numpy as jnp
from jax import lax
from jax.experimental import pallas as pl
from jax.experimental.pallas import tpu as pltpu
from jax.experimental.pallas import tpu_sc as plsc

N_NODES = 100000
N_ELEMS = 100000
EPAD = 100352
KEY_F = 131072
NW = 32
SLICE = EPAD // NW
CHUNKS = SLICE // 16
FILLB = 784
ROUNDS = 7
CH = EPAD // ROUNDS
RED = CH // 16
NB = 448
NBCH = NB // 16
SLOTS = 16

_mesh = plsc.VectorSubcoreMesh(core_axis_name="c", subcore_axis_name="s")
_sc_params = pltpu.CompilerParams(needs_layout_passes=False)


@functools.partial(
    pl.kernel,
    out_type=(
        jax.ShapeDtypeStruct((2 * EPAD,), jnp.int32),
        jax.ShapeDtypeStruct((2 * EPAD,), jnp.int32),
    ),
    mesh=_mesh,
    compiler_params=_sc_params,
    scratch_types=[
        pltpu.VMEM((EPAD,), jnp.int32),
        pltpu.VMEM((FILLB * 4,), jnp.int32),
        pltpu.VMEM((FILLB * 4,), jnp.int32),
        pltpu.VMEM((FILLB * 4,), jnp.float32),
        pltpu.VMEM_SHARED((16 * CH,), jnp.int32),
        pltpu.VMEM((RED,), jnp.int32),
        pltpu.VMEM((RED,), jnp.int32),
    ],
)
def _s1(eid_hbm, isa_hbm, mask_hbm, mka_hbm, mkb_hbm,
        tab, eid_b, isa_b, mask_b, spm, rbuf, racc):
    c = lax.axis_index("c")
    s = lax.axis_index("s")
    node_base = (c * 16 + s) * SLICE
    iota = lax.iota(jnp.int32, 16)
    neg1 = jnp.full((16,), -1, jnp.int32)

    for out_ref, want in ((mka_hbm, 1), (mkb_hbm, 0)):
        def init_body(j, _):
            tab[pl.ds(j * 64, 16)] = neg1
            tab[pl.ds(j * 64 + 16, 16)] = neg1
            tab[pl.ds(j * 64 + 32, 16)] = neg1
            tab[pl.ds(j * 64 + 48, 16)] = neg1
            return 0
        lax.fori_loop(0, EPAD // 64, init_body, 0)

        for b in range(SLICE // FILLB):
            fb = jnp.minimum(node_base + b * FILLB, N_NODES - FILLB)
            pltpu.sync_copy(eid_hbm.at[pl.ds(fb * 4, FILLB * 4)], eid_b)
            pltpu.sync_copy(isa_hbm.at[pl.ds(fb * 4, FILLB * 4)], isa_b)
            pltpu.sync_copy(mask_hbm.at[pl.ds(fb * 4, FILLB * 4)], mask_b)

            def fill_body(k, _):
                sl = pl.ds(k * 16, 16)
                g = k * 16 + iota
                eidv = eid_b[sl]
                valid = (mask_b[sl] > 0.5) & (isa_b[sl] == want)
                key = (g & 3) * KEY_F + (fb + (g >> 2))
                cur = plsc.load_gather(tab, [eidv])
                plsc.store_scatter(tab, [eidv], jnp.maximum(cur, key),
                                   mask=valid)
                return 0
            lax.fori_loop(0, FILLB * 4 // 16, fill_body, 0)

        for r in range(ROUNDS):
            pltpu.sync_copy(tab.at[pl.ds(r * CH, CH)],
                            spm.at[pl.ds(s * CH, CH)])
            plsc.subcore_barrier()
            myoff = s * RED
            pltpu.sync_copy(spm.at[pl.ds(myoff, RED)], racc)
            for t in range(1, 16):
                pltpu.sync_copy(spm.at[pl.ds(t * CH + myoff, RED)], rbuf)

                def red_body(j, _):
                    sl = pl.ds(j * 16, 16)
                    racc[sl] = jnp.maximum(racc[sl], rbuf[sl])
                    return 0
                lax.fori_loop(0, RED // 16, red_body, 0)
            pltpu.sync_copy(
                racc, out_ref.at[pl.ds(c * EPAD + r * CH + myoff, RED)])
            plsc.subcore_barrier()


@functools.partial(
    pl.kernel,
    out_type=jax.ShapeDtypeStruct((2 * 16 * 256,), jnp.float32),
    mesh=_mesh,
    compiler_params=_sc_params,
    scratch_types=(
        [pltpu.VMEM((SLICE,), jnp.int32) for _ in range(2)]
        + [pltpu.VMEM((SLICE,), jnp.int32)]
        + [pltpu.VMEM((2 * SLICE,), jnp.int32)]
        + [pltpu.VMEM((3 * SLICE,), jnp.float32)]
        + [pltpu.VMEM((SLICE,), jnp.float32) for _ in range(4)]
        + [pltpu.VMEM((SLICE,), jnp.int32) for _ in range(12)]
        + [pltpu.VMEM((SLICE,), jnp.float32) for _ in range(12)]
        + [pltpu.VMEM((NB * 15,), jnp.float32),
           pltpu.VMEM((NB * 4,), jnp.float32),
           pltpu.VMEM((NB,), jnp.float32),
           pltpu.VMEM((NB,), jnp.float32),
           pltpu.VMEM((NB * 3,), jnp.float32)]
        + [pltpu.VMEM((256,), jnp.float32),
           pltpu.VMEM_SHARED((16 * 256,), jnp.float32),
           pltpu.SemaphoreType.DMA]
    ),
)
def _s2(mka_hbm, mkb_hbm, pred_hbm, conn_hbm, dirs_hbm, len_hbm,
        pe_hbm, pa_hbm, pi_hbm, fm_hbm, bcd_hbm, bcr_hbm, fe_hbm,
        part_hbm,
        mka, mkb, tmp, conb, dirb, lb, peb, pab, pib,
        ixa0, ixa1, ixa2, ixb0, ixb1, ixb2,
        ixda0, ixda1, ixda2, ixdb0, ixdb1, ixdb2,
        ga0, ga1, ga2, gb0, gb1, gb2,
        gda0, gda1, gda2, gdb0, gdb1, gdb2,
        pnb, fmb, bcdb, bcrb, feb,
        obuf, spmf, sem):
    c = lax.axis_index("c")
    s = lax.axis_index("s")
    wid = c * 16 + s
    lo = wid * SLICE
    eb = jnp.minimum(lo, N_ELEMS - SLICE)
    iota = lax.iota(jnp.int32, 16)

    pltpu.sync_copy(mka_hbm.at[pl.ds(eb, SLICE)], mka)
    pltpu.sync_copy(mka_hbm.at[pl.ds(EPAD + eb, SLICE)], tmp)

    def fold_a(k, x):
        sl = pl.ds(k * 16, 16)
        mka[sl] = jnp.maximum(mka[sl], tmp[sl])
        return x
    lax.fori_loop(0, CHUNKS, fold_a, 0)
    pltpu.sync_copy(mkb_hbm.at[pl.ds(eb, SLICE)], mkb)
    pltpu.sync_copy(mkb_hbm.at[pl.ds(EPAD + eb, SLICE)], tmp)

    def fold_b(k, x):
        sl = pl.ds(k * 16, 16)
        mkb[sl] = jnp.maximum(mkb[sl], tmp[sl])
        return x
    lax.fori_loop(0, CHUNKS, fold_b, 0)

    pltpu.sync_copy(conn_hbm.at[pl.ds(2 * eb, 2 * SLICE)], conb)
    pltpu.sync_copy(dirs_hbm.at[pl.ds(3 * eb, 3 * SLICE)], dirb)
    pltpu.sync_copy(len_hbm.at[pl.ds(eb, SLICE)], lb)
    pltpu.sync_copy(pe_hbm.at[pl.ds(eb, SLICE)], peb)
    pltpu.sync_copy(pa_hbm.at[pl.ds(eb, SLICE)], pab)
    pltpu.sync_copy(pi_hbm.at[pl.ds(eb, SLICE)], pib)

    ixa = (ixa0, ixa1, ixa2)
    ixb = (ixb0, ixb1, ixb2)
    ixda = (ixda0, ixda1, ixda2)
    ixdb = (ixdb0, ixdb1, ixdb2)

    def idx_body(k, _):
        sl = pl.ds(k * 16, 16)
        gid = eb + k * 16 + iota
        spread = gid * 14
        a = mka[sl]
        b = mkb[sl]
        rowa = 15 * (a & (KEY_F - 1)) + 3 * (a >> 17) + 3
        rowb = 15 * (b & (KEY_F - 1)) + 3 * (b >> 17) + 3
        e2 = (k * 16 + iota) * 2
        e3 = (k * 16 + iota) * 3
        cna = plsc.load_gather(conb, [e2])
        cnb_ = plsc.load_gather(conb, [e2 + 1])
        for comp in range(3):
            ixa[comp][sl] = jnp.where(a >= 0, rowa + comp, spread)
            ixb[comp][sl] = jnp.where(b >= 0, rowb + comp, spread)
            ixda[comp][sl] = 15 * cna + comp
            ixdb[comp][sl] = 15 * cnb_ + comp
        return 0
    lax.fori_loop(0, CHUNKS, idx_body, 0)

    copies = []
    for ix, dst in ((ixa0, ga0), (ixa1, ga1), (ixa2, ga2),
                    (ixb0, gb0), (ixb1, gb1), (ixb2, gb2),
                    (ixda0, gda0), (ixda1, gda1), (ixda2, gda2),
                    (ixdb0, gdb0), (ixdb1, gdb1), (ixdb2, gdb2)):
        copies.append(pltpu.async_copy(pred_hbm.at[ix], dst, sem))

    zero = jnp.zeros((16,), jnp.float32)
    nacc = [zero] * 11
    node_base = wid * SLICE
    for blk in range(SLICE // NB):
        nlo = node_base + blk * NB
        nb0 = jnp.minimum(nlo, N_NODES - NB)
        pltpu.sync_copy(pred_hbm.at[pl.ds(nb0 * 15, NB * 15)], pnb)
        pltpu.sync_copy(fm_hbm.at[pl.ds(nb0 * 4, NB * 4)], fmb)
        pltpu.sync_copy(bcd_hbm.at[pl.ds(nb0, NB)], bcdb)
        pltpu.sync_copy(bcr_hbm.at[pl.ds(nb0, NB)], bcrb)
        pltpu.sync_copy(fe_hbm.at[pl.ds(nb0 * 3, NB * 3)], feb)

        def node_body(k, carry):
            (aq0, aq1, acf, aq2, aq3, acff, aq4, aq5, aq6, acsd, acsr) = carry
            sl = pl.ds(k * 16, 16)
            lane = k * 16 + iota
            nid = nb0 + lane
            m = ((nid >= nlo) & (nid < N_NODES)).astype(jnp.float32)
            p15 = lane * 15
            cols = [plsc.load_gather(pnb, [p15 + cc]) for cc in range(15)]
            bcd = bcdb[sl]
            bcr = bcrb[sl]
            free = m * (bcd < 0.5).astype(jnp.float32)
            supd = m * (bcd > 0.5).astype(jnp.float32)
            supr = m * (bcr > 0.5).astype(jnp.float32)
            f3 = lane * 3
            r0 = cols[3] + cols[6] + cols[9] + cols[12] \
                - plsc.load_gather(feb, [f3])
            r1 = cols[4] + cols[7] + cols[10] + cols[13] \
                - plsc.load_gather(feb, [f3 + 1])
            r2 = cols[5] + cols[8] + cols[11] + cols[14] \
                - plsc.load_gather(feb, [f3 + 2])
            aq0 = aq0 + free * (r0 * r0 + r1 * r1)
            aq1 = aq1 + free * (r2 * r2)
            acf = acf + free
            f4 = lane * 4
            for f in range(4):
                ffree = m * (plsc.load_gather(fmb, [f4 + f]) < 0.5).astype(
                    jnp.float32)
                g0 = cols[3 + 3 * f]
                g1 = cols[4 + 3 * f]
                g2 = cols[5 + 3 * f]
                aq2 = aq2 + ffree * (g0 * g0 + g1 * g1)
                aq3 = aq3 + ffree * (g2 * g2)
                acff = acff + ffree
            aq4 = aq4 + supd * cols[0] * cols[0]
            aq5 = aq5 + supd * cols[1] * cols[1]
            aq6 = aq6 + supr * cols[2] * cols[2]
            acsd = acsd + supd
            acsr = acsr + supr
            return (aq0, aq1, acf, aq2, aq3, acff, aq4, aq5, aq6, acsd, acsr)

        nacc = list(lax.fori_loop(0, NBCH, node_body, tuple(nacc)))

    for cp in copies:
        cp.wait()

    def phys_body(k, carry):
        acc_n, acc_m, acc_v = carry
        sl = pl.ds(k * 16, 16)
        e3 = (k * 16 + iota) * 3
        eidv = eb + k * 16 + iota
        oka = (mka[sl] >= 0).astype(jnp.float32)
        okb = (mkb[sl] >= 0).astype(jnp.float32)

        fa0 = ga0[sl] * oka
        fa1 = ga1[sl] * oka
        fa2 = ga2[sl] * oka
        fb0 = gb0[sl] * okb
        fb1 = gb1[sl] * okb
        fb2 = gb2[sl] * okb
        da0 = gda0[sl]
        da1 = gda1[sl]
        da2 = gda2[sl]
        db0 = gdb0[sl]
        db1 = gdb1[sl]
        db2 = gdb2[sl]

        cs = plsc.load_gather(dirb, [e3])
        sn = plsc.load_gather(dirb, [e3 + 2])
        lv = lb[sl]
        ea = peb[sl] * pab[sl]
        ei = peb[sl] * pib[sl]

        u_a = da0 * cs + da1 * sn
        w_a = -da0 * sn + da1 * cs
        t_a = da2
        u_b = db0 * cs + db1 * sn
        w_b = -db0 * sn + db1 * cs
        t_b = db2
        fra0 = fa0 * cs + fa1 * sn
        fra1 = -fa0 * sn + fa1 * cs
        fra2 = fa2
        frb0 = fb0 * cs + fb1 * sn
        frb1 = -fb0 * sn + fb1 * cs
        frb2 = fb2

        l2 = lv * lv
        l3 = l2 * lv
        n_sf = ea * (u_b - u_a) / lv
        m_a = ei / l2 * (-6.0 * w_a - 4.0 * lv * t_a + 6.0 * w_b - 2.0 * lv * t_b)
        m_b = ei / l2 * (6.0 * w_a + 2.0 * lv * t_a - 6.0 * w_b + 4.0 * lv * t_b)
        v_sf = ei / l3 * (12.0 * w_a + 6.0 * lv * t_a - 12.0 * w_b + 6.0 * lv * t_b)

        m = ((eidv >= lo) & (eidv < N_ELEMS)).astype(jnp.float32)
        rn0 = fra0 + n_sf
        rn1 = frb0 - n_sf
        rm0 = fra2 + m_a
        rm1 = frb2 - m_b
        rv0 = fra1 + v_sf
        rv1 = frb1 - v_sf
        acc_n = acc_n + m * (rn0 * rn0 + rn1 * rn1)
        acc_m = acc_m + m * (rm0 * rm0 + rm1 * rm1)
        acc_v = acc_v + m * (rv0 * rv0 + rv1 * rv1)
        return acc_n, acc_m, acc_v

    acc_n, acc_m, acc_v = lax.fori_loop(0, CHUNKS, phys_body,
                                        (zero, zero, zero))

    slots = [acc_n, acc_m, acc_v] + nacc
    for i, v in enumerate(slots):
        obuf[pl.ds(i * SLOTS, SLOTS)] = v
    obuf[pl.ds(14 * SLOTS, SLOTS)] = zero
    obuf[pl.ds(15 * SLOTS, SLOTS)] = zero
    pltpu.sync_copy(obuf, spmf.at[pl.ds(s * 256, 256)])
    plsc.subcore_barrier()

    @pl.when(s == 0)
    def _():
        pltpu.sync_copy(spmf, part_hbm.at[pl.ds(c * 16 * 256, 16 * 256)])


def _final_tc_body(ep_ref, fc_ref, mc_ref, uc_ref, tc_ref, out_ref):
    ep = ep_ref[...]
    fc = fc_ref[0, 0]
    mc = mc_ref[0, 0]
    uc = uc_ref[0, 0]
    th = tc_ref[0, 0]
    fc2 = fc * fc
    mc2 = mc * mc

    def slot(i):
        return jnp.sum(ep[:, 16 * i:16 * (i + 1)])

    l_eq = (slot(3) / fc2 + slot(4) / mc2) / jnp.maximum(slot(5), 1.0)
    l_free = (slot(6) / fc2 + slot(7) / mc2) / jnp.maximum(slot(8) * 3.0, 1.0)
    l_sup = ((slot(9) + slot(10)) / (uc * uc) / jnp.maximum(slot(12), 1.0)
             + slot(11) / (th * th) / jnp.maximum(slot(13), 1.0))
    e_cnt = float(N_ELEMS)
    total = (l_eq + l_free + l_sup
             + slot(0) / fc2 / e_cnt + slot(1) / mc2 / e_cnt
             + slot(2) / fc2 / e_cnt)
    out_ref[...] = jnp.reshape(total, (1, 1))


def _final_tc(ep, fc, mc, uc, th):
    return pl.pallas_call(
        _final_tc_body,
        in_specs=[
            pl.BlockSpec((NW, 256), lambda: (0, 0)),
            pl.BlockSpec((1, 1), lambda: (0, 0)),
            pl.BlockSpec((1, 1), lambda: (0, 0)),
            pl.BlockSpec((1, 1), lambda: (0, 0)),
            pl.BlockSpec((1, 1), lambda: (0, 0)),
        ],
        out_specs=pl.BlockSpec((1, 1), lambda: (0, 0)),
        out_shape=jax.ShapeDtypeStruct((1, 1), jnp.float32),
    )(ep, fc, mc, uc, th)


def kernel(pred, connectivity, face_element_id, face_is_A_end, face_mask,
           F_ext, bc_disp, bc_rot, elem_directions, elem_lengths,
           prop_E, prop_A, prop_I22, F_c, M_c, u_c, theta_c):
    mka, mkb = _s1(face_element_id.reshape(-1),
                   face_is_A_end.reshape(-1),
                   face_mask.reshape(-1))
    part = _s2(mka, mkb, pred.reshape(-1), connectivity.reshape(-1),
               elem_directions.reshape(-1), elem_lengths,
               prop_E, prop_A, prop_I22, face_mask.reshape(-1),
               bc_disp.reshape(-1), bc_rot.reshape(-1), F_ext.reshape(-1))
    out = _final_tc(part.reshape(NW, 256), F_c.reshape(1, 1),
                    M_c.reshape(1, 1), u_c.reshape(1, 1),
                    theta_c.reshape(1, 1))
    return out[0, 0]


# speedup vs baseline: 1.4633x; 1.0116x over previous
import functools

import jax
import jax.numpy as jnp
from jax import lax
from jax.experimental import pallas as pl
from jax.experimental.pallas import tpu as pltpu
from jax.experimental.pallas import tpu_sc as plsc

N_NODES = 100000
N_ELEMS = 100000
EPAD = 100352
KEY_F = 131072
NW = 32
SLICE = EPAD // NW
CHUNKS = SLICE // 16
FILLB = 784
ROUNDS = 7
CH = EPAD // ROUNDS
RED = CH // 16
NB = 448
NBCH = NB // 16
SLOTS = 16

_mesh = plsc.VectorSubcoreMesh(core_axis_name="c", subcore_axis_name="s")
_sc_params = pltpu.CompilerParams(needs_layout_passes=False)


@functools.partial(
    pl.kernel,
    out_type=(
        jax.ShapeDtypeStruct((2 * EPAD,), jnp.int32),
        jax.ShapeDtypeStruct((2 * EPAD,), jnp.int32),
    ),
    mesh=_mesh,
    compiler_params=_sc_params,
    scratch_types=[
        pltpu.VMEM((EPAD,), jnp.int32),
        pltpu.VMEM((FILLB * 4,), jnp.int32),
        pltpu.VMEM((FILLB * 4,), jnp.int32),
        pltpu.VMEM((FILLB * 4,), jnp.float32),
        pltpu.VMEM_SHARED((16 * CH,), jnp.int32),
        pltpu.VMEM((RED,), jnp.int32),
        pltpu.VMEM((RED,), jnp.int32),
    ],
)
def _s1(eid_hbm, isa_hbm, mask_hbm, mka_hbm, mkb_hbm,
        tab, eid_b, isa_b, mask_b, spm, rbuf, racc):
    c = lax.axis_index("c")
    s = lax.axis_index("s")
    node_base = (c * 16 + s) * SLICE
    iota = lax.iota(jnp.int32, 16)
    neg1 = jnp.full((16,), -1, jnp.int32)

    for out_ref, want in ((mka_hbm, 1), (mkb_hbm, 0)):
        def init_body(j, _):
            tab[pl.ds(j * 64, 16)] = neg1
            tab[pl.ds(j * 64 + 16, 16)] = neg1
            tab[pl.ds(j * 64 + 32, 16)] = neg1
            tab[pl.ds(j * 64 + 48, 16)] = neg1
            return 0
        lax.fori_loop(0, EPAD // 64, init_body, 0)

        for b in range(SLICE // FILLB):
            fb = jnp.minimum(node_base + b * FILLB, N_NODES - FILLB)
            pltpu.sync_copy(eid_hbm.at[pl.ds(fb * 4, FILLB * 4)], eid_b)
            pltpu.sync_copy(isa_hbm.at[pl.ds(fb * 4, FILLB * 4)], isa_b)
            pltpu.sync_copy(mask_hbm.at[pl.ds(fb * 4, FILLB * 4)], mask_b)

            def fill_body(k, _):
                for u in range(2):
                    sl = pl.ds(k * 32 + u * 16, 16)
                    g = k * 32 + u * 16 + iota
                    eidv = eid_b[sl]
                    valid = (mask_b[sl] > 0.5) & (isa_b[sl] == want)
                    key = (g & 3) * KEY_F + (fb + (g >> 2))
                    cur = plsc.load_gather(tab, [eidv])
                    plsc.store_scatter(tab, [eidv], jnp.maximum(cur, key),
                                       mask=valid)
                return 0
            lax.fori_loop(0, FILLB * 4 // 32, fill_body, 0)

        for r in range(ROUNDS):
            pltpu.sync_copy(tab.at[pl.ds(r * CH, CH)],
                            spm.at[pl.ds(s * CH, CH)])
            plsc.subcore_barrier()
            myoff = s * RED
            pltpu.sync_copy(spm.at[pl.ds(myoff, RED)], racc)
            for t in range(1, 16):
                pltpu.sync_copy(spm.at[pl.ds(t * CH + myoff, RED)], rbuf)

                def red_body(j, _):
                    for u in range(4):
                        sl = pl.ds(j * 64 + u * 16, 16)
                        racc[sl] = jnp.maximum(racc[sl], rbuf[sl])
                    return 0
                lax.fori_loop(0, RED // 64, red_body, 0)
            pltpu.sync_copy(
                racc, out_ref.at[pl.ds(c * EPAD + r * CH + myoff, RED)])
            plsc.subcore_barrier()


@functools.partial(
    pl.kernel,
    out_type=jax.ShapeDtypeStruct((2 * 16 * 256,), jnp.float32),
    mesh=_mesh,
    compiler_params=_sc_params,
    scratch_types=(
        [pltpu.VMEM((SLICE,), jnp.int32) for _ in range(2)]
        + [pltpu.VMEM((SLICE,), jnp.int32)]
        + [pltpu.VMEM((2 * SLICE,), jnp.int32)]
        + [pltpu.VMEM((3 * SLICE,), jnp.float32)]
        + [pltpu.VMEM((SLICE,), jnp.float32) for _ in range(4)]
        + [pltpu.VMEM((SLICE,), jnp.int32) for _ in range(12)]
        + [pltpu.VMEM((SLICE,), jnp.float32) for _ in range(12)]
        + [pltpu.VMEM((NB * 15,), jnp.float32),
           pltpu.VMEM((NB * 4,), jnp.float32),
           pltpu.VMEM((NB,), jnp.float32),
           pltpu.VMEM((NB,), jnp.float32),
           pltpu.VMEM((NB * 3,), jnp.float32)]
        + [pltpu.VMEM((256,), jnp.float32),
           pltpu.VMEM_SHARED((16 * 256,), jnp.float32),
           pltpu.SemaphoreType.DMA,
           pltpu.SemaphoreType.DMA]
    ),
)
def _s2(mka_hbm, mkb_hbm, pred_hbm, conn_hbm, dirs_hbm, len_hbm,
        pe_hbm, pa_hbm, pi_hbm, fm_hbm, bcd_hbm, bcr_hbm, fe_hbm,
        part_hbm,
        mka, mkb, tmp, conb, dirb, lb, peb, pab, pib,
        ixa0, ixa1, ixa2, ixb0, ixb1, ixb2,
        ixda0, ixda1, ixda2, ixdb0, ixdb1, ixdb2,
        ga0, ga1, ga2, gb0, gb1, gb2,
        gda0, gda1, gda2, gdb0, gdb1, gdb2,
        pnb, fmb, bcdb, bcrb, feb,
        obuf, spmf, sem, sem2):
    c = lax.axis_index("c")
    s = lax.axis_index("s")
    wid = c * 16 + s
    lo = wid * SLICE
    eb = jnp.minimum(lo, N_ELEMS - SLICE)
    iota = lax.iota(jnp.int32, 16)

    stage = [
        pltpu.async_copy(mka_hbm.at[pl.ds(eb, SLICE)], mka, sem2),
        pltpu.async_copy(mka_hbm.at[pl.ds(EPAD + eb, SLICE)], tmp, sem2),
        pltpu.async_copy(conn_hbm.at[pl.ds(2 * eb, 2 * SLICE)], conb, sem2),
        pltpu.async_copy(dirs_hbm.at[pl.ds(3 * eb, 3 * SLICE)], dirb, sem2),
        pltpu.async_copy(len_hbm.at[pl.ds(eb, SLICE)], lb, sem2),
        pltpu.async_copy(pe_hbm.at[pl.ds(eb, SLICE)], peb, sem2),
        pltpu.async_copy(pa_hbm.at[pl.ds(eb, SLICE)], pab, sem2),
        pltpu.async_copy(pi_hbm.at[pl.ds(eb, SLICE)], pib, sem2),
    ]
    for cp in stage:
        cp.wait()

    def fold_a(k, x):
        sl = pl.ds(k * 16, 16)
        mka[sl] = jnp.maximum(mka[sl], tmp[sl])
        return x
    lax.fori_loop(0, CHUNKS, fold_a, 0)
    pltpu.sync_copy(mkb_hbm.at[pl.ds(eb, SLICE)], mkb)
    pltpu.sync_copy(mkb_hbm.at[pl.ds(EPAD + eb, SLICE)], tmp)

    def fold_b(k, x):
        sl = pl.ds(k * 16, 16)
        mkb[sl] = jnp.maximum(mkb[sl], tmp[sl])
        return x
    lax.fori_loop(0, CHUNKS, fold_b, 0)

    ixa = (ixa0, ixa1, ixa2)
    ixb = (ixb0, ixb1, ixb2)
    ixda = (ixda0, ixda1, ixda2)
    ixdb = (ixdb0, ixdb1, ixdb2)

    def idx_body(k, _):
        sl = pl.ds(k * 16, 16)
        gid = eb + k * 16 + iota
        spread = gid * 14
        a = mka[sl]
        b = mkb[sl]
        rowa = 15 * (a & (KEY_F - 1)) + 3 * (a >> 17) + 3
        rowb = 15 * (b & (KEY_F - 1)) + 3 * (b >> 17) + 3
        e2 = (k * 16 + iota) * 2
        cna = plsc.load_gather(conb, [e2])
        cnb_ = plsc.load_gather(conb, [e2 + 1])
        for comp in range(3):
            ixa[comp][sl] = jnp.where(a >= 0, rowa + comp, spread)
            ixb[comp][sl] = jnp.where(b >= 0, rowb + comp, spread)
            ixda[comp][sl] = 15 * cna + comp
            ixdb[comp][sl] = 15 * cnb_ + comp
        return 0
    lax.fori_loop(0, CHUNKS, idx_body, 0)

    copies = []
    for ix, dst in ((ixa0, ga0), (ixa1, ga1), (ixa2, ga2),
                    (ixb0, gb0), (ixb1, gb1), (ixb2, gb2),
                    (ixda0, gda0), (ixda1, gda1), (ixda2, gda2),
                    (ixdb0, gdb0), (ixdb1, gdb1), (ixdb2, gdb2)):
        copies.append(pltpu.async_copy(pred_hbm.at[ix], dst, sem))

    zero = jnp.zeros((16,), jnp.float32)
    nacc = [zero] * 11
    node_base = wid * SLICE
    for blk in range(SLICE // NB):
        nlo = node_base + blk * NB
        nb0 = jnp.minimum(nlo, N_NODES - NB)
        nstage = [
            pltpu.async_copy(pred_hbm.at[pl.ds(nb0 * 15, NB * 15)], pnb,
                             sem2),
            pltpu.async_copy(fm_hbm.at[pl.ds(nb0 * 4, NB * 4)], fmb, sem2),
            pltpu.async_copy(bcd_hbm.at[pl.ds(nb0, NB)], bcdb, sem2),
            pltpu.async_copy(bcr_hbm.at[pl.ds(nb0, NB)], bcrb, sem2),
            pltpu.async_copy(fe_hbm.at[pl.ds(nb0 * 3, NB * 3)], feb, sem2),
        ]
        for cp in nstage:
            cp.wait()

        def node_body(k, carry):
            (aq0, aq1, acf, aq2, aq3, acff, aq4, aq5, aq6, acsd, acsr) = carry
            sl = pl.ds(k * 16, 16)
            lane = k * 16 + iota
            nid = nb0 + lane
            m = ((nid >= nlo) & (nid < N_NODES)).astype(jnp.float32)
            p15 = lane * 15
            cols = [plsc.load_gather(pnb, [p15 + cc]) for cc in range(15)]
            bcd = bcdb[sl]
            bcr = bcrb[sl]
            free = m * (bcd < 0.5).astype(jnp.float32)
            supd = m * (bcd > 0.5).astype(jnp.float32)
            supr = m * (bcr > 0.5).astype(jnp.float32)
            f3 = lane * 3
            r0 = cols[3] + cols[6] + cols[9] + cols[12] \
                - plsc.load_gather(feb, [f3])
            r1 = cols[4] + cols[7] + cols[10] + cols[13] \
                - plsc.load_gather(feb, [f3 + 1])
            r2 = cols[5] + cols[8] + cols[11] + cols[14] \
                - plsc.load_gather(feb, [f3 + 2])
            aq0 = aq0 + free * (r0 * r0 + r1 * r1)
            aq1 = aq1 + free * (r2 * r2)
            acf = acf + free
            f4 = lane * 4
            for f in range(4):
                ffree = m * (plsc.load_gather(fmb, [f4 + f]) < 0.5).astype(
                    jnp.float32)
                g0 = cols[3 + 3 * f]
                g1 = cols[4 + 3 * f]
                g2 = cols[5 + 3 * f]
                aq2 = aq2 + ffree * (g0 * g0 + g1 * g1)
                aq3 = aq3 + ffree * (g2 * g2)
                acff = acff + ffree
            aq4 = aq4 + supd * cols[0] * cols[0]
            aq5 = aq5 + supd * cols[1] * cols[1]
            aq6 = aq6 + supr * cols[2] * cols[2]
            acsd = acsd + supd
            acsr = acsr + supr
            return (aq0, aq1, acf, aq2, aq3, acff, aq4, aq5, aq6, acsd, acsr)

        nacc = list(lax.fori_loop(0, NBCH, node_body, tuple(nacc)))

    for cp in copies:
        cp.wait()

    def phys_body(k, carry):
        acc_n, acc_m, acc_v = carry
        sl = pl.ds(k * 16, 16)
        e3 = (k * 16 + iota) * 3
        eidv = eb + k * 16 + iota
        oka = (mka[sl] >= 0).astype(jnp.float32)
        okb = (mkb[sl] >= 0).astype(jnp.float32)

        fa0 = ga0[sl] * oka
        fa1 = ga1[sl] * oka
        fa2 = ga2[sl] * oka
        fb0 = gb0[sl] * okb
        fb1 = gb1[sl] * okb
        fb2 = gb2[sl] * okb
        da0 = gda0[sl]
        da1 = gda1[sl]
        da2 = gda2[sl]
        db0 = gdb0[sl]
        db1 = gdb1[sl]
        db2 = gdb2[sl]

        cs = plsc.load_gather(dirb, [e3])
        sn = plsc.load_gather(dirb, [e3 + 2])
        lv = lb[sl]
        ea = peb[sl] * pab[sl]
        ei = peb[sl] * pib[sl]

        u_a = da0 * cs + da1 * sn
        w_a = -da0 * sn + da1 * cs
        t_a = da2
        u_b = db0 * cs + db1 * sn
        w_b = -db0 * sn + db1 * cs
        t_b = db2
        fra0 = fa0 * cs + fa1 * sn
        fra1 = -fa0 * sn + fa1 * cs
        fra2 = fa2
        frb0 = fb0 * cs + fb1 * sn
        frb1 = -fb0 * sn + fb1 * cs
        frb2 = fb2

        l2 = lv * lv
        l3 = l2 * lv
        n_sf = ea * (u_b - u_a) / lv
        m_a = ei / l2 * (-6.0 * w_a - 4.0 * lv * t_a + 6.0 * w_b - 2.0 * lv * t_b)
        m_b = ei / l2 * (6.0 * w_a + 2.0 * lv * t_a - 6.0 * w_b + 4.0 * lv * t_b)
        v_sf = ei / l3 * (12.0 * w_a + 6.0 * lv * t_a - 12.0 * w_b + 6.0 * lv * t_b)

        m = ((eidv >= lo) & (eidv < N_ELEMS)).astype(jnp.float32)
        rn0 = fra0 + n_sf
        rn1 = frb0 - n_sf
        rm0 = fra2 + m_a
        rm1 = frb2 - m_b
        rv0 = fra1 + v_sf
        rv1 = frb1 - v_sf
        acc_n = acc_n + m * (rn0 * rn0 + rn1 * rn1)
        acc_m = acc_m + m * (rm0 * rm0 + rm1 * rm1)
        acc_v = acc_v + m * (rv0 * rv0 + rv1 * rv1)
        return acc_n, acc_m, acc_v

    acc_n, acc_m, acc_v = lax.fori_loop(0, CHUNKS, phys_body,
                                        (zero, zero, zero))

    slots = [acc_n, acc_m, acc_v] + nacc
    for i, v in enumerate(slots):
        obuf[pl.ds(i * SLOTS, SLOTS)] = v
    obuf[pl.ds(14 * SLOTS, SLOTS)] = zero
    obuf[pl.ds(15 * SLOTS, SLOTS)] = zero
    pltpu.sync_copy(obuf, spmf.at[pl.ds(s * 256, 256)])
    plsc.subcore_barrier()

    @pl.when(s == 0)
    def _():
        pltpu.sync_copy(spmf, part_hbm.at[pl.ds(c * 16 * 256, 16 * 256)])


def _final_tc_body(ep_ref, fc_ref, mc_ref, uc_ref, tc_ref, out_ref):
    ep = ep_ref[...]
    fc = fc_ref[0, 0]
    mc = mc_ref[0, 0]
    uc = uc_ref[0, 0]
    th = tc_ref[0, 0]
    fc2 = fc * fc
    mc2 = mc * mc

    def slot(i):
        return jnp.sum(ep[:, 16 * i:16 * (i + 1)])

    l_eq = (slot(3) / fc2 + slot(4) / mc2) / jnp.maximum(slot(5), 1.0)
    l_free = (slot(6) / fc2 + slot(7) / mc2) / jnp.maximum(slot(8) * 3.0, 1.0)
    l_sup = ((slot(9) + slot(10)) / (uc * uc) / jnp.maximum(slot(12), 1.0)
             + slot(11) / (th * th) / jnp.maximum(slot(13), 1.0))
    e_cnt = float(N_ELEMS)
    total = (l_eq + l_free + l_sup
             + slot(0) / fc2 / e_cnt + slot(1) / mc2 / e_cnt
             + slot(2) / fc2 / e_cnt)
    out_ref[...] = jnp.reshape(total, (1, 1))


def _final_tc(ep, fc, mc, uc, th):
    return pl.pallas_call(
        _final_tc_body,
        in_specs=[
            pl.BlockSpec((NW, 256), lambda: (0, 0)),
            pl.BlockSpec((1, 1), lambda: (0, 0)),
            pl.BlockSpec((1, 1), lambda: (0, 0)),
            pl.BlockSpec((1, 1), lambda: (0, 0)),
            pl.BlockSpec((1, 1), lambda: (0, 0)),
        ],
        out_specs=pl.BlockSpec((1, 1), lambda: (0, 0)),
        out_shape=jax.ShapeDtypeStruct((1, 1), jnp.float32),
    )(ep, fc, mc, uc, th)


def kernel(pred, connectivity, face_element_id, face_is_A_end, face_mask,
           F_ext, bc_disp, bc_rot, elem_directions, elem_lengths,
           prop_E, prop_A, prop_I22, F_c, M_c, u_c, theta_c):
    mka, mkb = _s1(face_element_id.reshape(-1),
                   face_is_A_end.reshape(-1),
                   face_mask.reshape(-1))
    part = _s2(mka, mkb, pred.reshape(-1), connectivity.reshape(-1),
               elem_directions.reshape(-1), elem_lengths,
               prop_E, prop_A, prop_I22, face_mask.reshape(-1),
               bc_disp.reshape(-1), bc_rot.reshape(-1), F_ext.reshape(-1))
    out = _final_tc(part.reshape(NW, 256), F_c.reshape(1, 1),
                    M_c.reshape(1, 1), u_c.reshape(1, 1),
                    theta_c.reshape(1, 1))
    return out[0, 0]


# speedup vs baseline: 1.4634x; 1.0001x over previous
import functools

import jax
import jax.numpy as jnp
from jax import lax
from jax.experimental import pallas as pl
from jax.experimental.pallas import tpu as pltpu
from jax.experimental.pallas import tpu_sc as plsc

N_NODES = 100000
N_ELEMS = 100000
EPAD = 100352
KEY_F = 131072
NW = 32
SLICE = EPAD // NW
CHUNKS = SLICE // 16
FILLB = 784
ROUNDS = 7
CH = EPAD // ROUNDS
RED = CH // 16
NB = 448
NBCH = NB // 16
SLOTS = 16

_mesh = plsc.VectorSubcoreMesh(core_axis_name="c", subcore_axis_name="s")
_sc_params = pltpu.CompilerParams(needs_layout_passes=False)


@functools.partial(
    pl.kernel,
    out_type=(
        jax.ShapeDtypeStruct((2 * EPAD,), jnp.int32),
        jax.ShapeDtypeStruct((2 * EPAD,), jnp.int32),
    ),
    mesh=_mesh,
    compiler_params=_sc_params,
    scratch_types=[
        pltpu.VMEM((EPAD,), jnp.int32),
        pltpu.VMEM((FILLB * 4,), jnp.int32),
        pltpu.VMEM((FILLB * 4,), jnp.int32),
        pltpu.VMEM((FILLB * 4,), jnp.float32),
        pltpu.VMEM_SHARED((16 * CH,), jnp.int32),
        pltpu.VMEM((RED,), jnp.int32),
        pltpu.VMEM((RED,), jnp.int32),
    ],
)
def _s1(eid_hbm, isa_hbm, mask_hbm, mka_hbm, mkb_hbm,
        tab, eid_b, isa_b, mask_b, spm, rbuf, racc):
    c = lax.axis_index("c")
    s = lax.axis_index("s")
    node_base = (c * 16 + s) * SLICE
    iota = lax.iota(jnp.int32, 16)
    neg1 = jnp.full((16,), -1, jnp.int32)

    for out_ref, want in ((mka_hbm, 1), (mkb_hbm, 0)):
        def init_body(j, _):
            tab[pl.ds(j * 64, 16)] = neg1
            tab[pl.ds(j * 64 + 16, 16)] = neg1
            tab[pl.ds(j * 64 + 32, 16)] = neg1
            tab[pl.ds(j * 64 + 48, 16)] = neg1
            return 0
        lax.fori_loop(0, EPAD // 64, init_body, 0)

        for b in range(SLICE // FILLB):
            fb = jnp.minimum(node_base + b * FILLB, N_NODES - FILLB)
            pltpu.sync_copy(eid_hbm.at[pl.ds(fb * 4, FILLB * 4)], eid_b)
            pltpu.sync_copy(isa_hbm.at[pl.ds(fb * 4, FILLB * 4)], isa_b)
            pltpu.sync_copy(mask_hbm.at[pl.ds(fb * 4, FILLB * 4)], mask_b)

            def fill_body(k, _):
                for u in range(2):
                    sl = pl.ds(k * 32 + u * 16, 16)
                    g = k * 32 + u * 16 + iota
                    eidv = eid_b[sl]
                    valid = (mask_b[sl] > 0.5) & (isa_b[sl] == want)
                    key = (g & 3) * KEY_F + (fb + (g >> 2))
                    cur = plsc.load_gather(tab, [eidv])
                    plsc.store_scatter(tab, [eidv], jnp.maximum(cur, key),
                                       mask=valid)
                return 0
            lax.fori_loop(0, FILLB * 4 // 32, fill_body, 0)

        for r in range(ROUNDS):
            pltpu.sync_copy(tab.at[pl.ds(r * CH, CH)],
                            spm.at[pl.ds(s * CH, CH)])
            plsc.subcore_barrier()
            myoff = s * RED
            pltpu.sync_copy(spm.at[pl.ds(myoff, RED)], racc)
            for t in range(1, 16):
                pltpu.sync_copy(spm.at[pl.ds(t * CH + myoff, RED)], rbuf)

                def red_body(j, _):
                    for u in range(4):
                        sl = pl.ds(j * 64 + u * 16, 16)
                        racc[sl] = jnp.maximum(racc[sl], rbuf[sl])
                    return 0
                lax.fori_loop(0, RED // 64, red_body, 0)
            pltpu.sync_copy(
                racc, out_ref.at[pl.ds(c * EPAD + r * CH + myoff, RED)])
            plsc.subcore_barrier()


@functools.partial(
    pl.kernel,
    out_type=jax.ShapeDtypeStruct((2 * 16 * 256,), jnp.float32),
    mesh=_mesh,
    compiler_params=_sc_params,
    scratch_types=(
        [pltpu.VMEM((SLICE,), jnp.int32) for _ in range(2)]
        + [pltpu.VMEM((SLICE,), jnp.int32)]
        + [pltpu.VMEM((2 * SLICE,), jnp.int32)]
        + [pltpu.VMEM((3 * SLICE,), jnp.float32)]
        + [pltpu.VMEM((SLICE,), jnp.float32) for _ in range(4)]
        + [pltpu.VMEM((SLICE,), jnp.int32) for _ in range(12)]
        + [pltpu.VMEM((SLICE,), jnp.float32) for _ in range(12)]
        + [pltpu.VMEM((NB * 15,), jnp.float32),
           pltpu.VMEM((NB * 4,), jnp.float32),
           pltpu.VMEM((NB,), jnp.float32),
           pltpu.VMEM((NB,), jnp.float32),
           pltpu.VMEM((NB * 3,), jnp.float32)]
        + [pltpu.VMEM((256,), jnp.float32),
           pltpu.VMEM_SHARED((16 * 256,), jnp.float32),
           pltpu.SemaphoreType.DMA,
           pltpu.SemaphoreType.DMA]
    ),
)
def _s2(mka_hbm, mkb_hbm, pred_hbm, conn_hbm, dirs_hbm, len_hbm,
        pe_hbm, pa_hbm, pi_hbm, fm_hbm, bcd_hbm, bcr_hbm, fe_hbm,
        part_hbm,
        mka, mkb, tmp, conb, dirb, lb, peb, pab, pib,
        ixa0, ixa1, ixa2, ixb0, ixb1, ixb2,
        ixda0, ixda1, ixda2, ixdb0, ixdb1, ixdb2,
        ga0, ga1, ga2, gb0, gb1, gb2,
        gda0, gda1, gda2, gdb0, gdb1, gdb2,
        pnb, fmb, bcdb, bcrb, feb,
        obuf, spmf, sem, sem2):
    c = lax.axis_index("c")
    s = lax.axis_index("s")
    wid = c * 16 + s
    lo = wid * SLICE
    eb = jnp.minimum(lo, N_ELEMS - SLICE)
    iota = lax.iota(jnp.int32, 16)

    stage = [
        pltpu.async_copy(mka_hbm.at[pl.ds(eb, SLICE)], mka, sem2),
        pltpu.async_copy(mka_hbm.at[pl.ds(EPAD + eb, SLICE)], tmp, sem2),
        pltpu.async_copy(conn_hbm.at[pl.ds(2 * eb, 2 * SLICE)], conb, sem2),
        pltpu.async_copy(dirs_hbm.at[pl.ds(3 * eb, 3 * SLICE)], dirb, sem2),
        pltpu.async_copy(len_hbm.at[pl.ds(eb, SLICE)], lb, sem2),
        pltpu.async_copy(pe_hbm.at[pl.ds(eb, SLICE)], peb, sem2),
        pltpu.async_copy(pa_hbm.at[pl.ds(eb, SLICE)], pab, sem2),
        pltpu.async_copy(pi_hbm.at[pl.ds(eb, SLICE)], pib, sem2),
    ]
    for cp in stage:
        cp.wait()

    def fold_a(k, x):
        sl = pl.ds(k * 16, 16)
        mka[sl] = jnp.maximum(mka[sl], tmp[sl])
        return x
    lax.fori_loop(0, CHUNKS, fold_a, 0)
    pltpu.sync_copy(mkb_hbm.at[pl.ds(eb, SLICE)], mkb)
    pltpu.sync_copy(mkb_hbm.at[pl.ds(EPAD + eb, SLICE)], tmp)

    def fold_b(k, x):
        sl = pl.ds(k * 16, 16)
        mkb[sl] = jnp.maximum(mkb[sl], tmp[sl])
        return x
    lax.fori_loop(0, CHUNKS, fold_b, 0)

    ixa = (ixa0, ixa1, ixa2)
    ixb = (ixb0, ixb1, ixb2)
    ixda = (ixda0, ixda1, ixda2)
    ixdb = (ixdb0, ixdb1, ixdb2)

    def idx_body(k, _):
        sl = pl.ds(k * 16, 16)
        gid = eb + k * 16 + iota
        spread = gid * 14
        a = mka[sl]
        b = mkb[sl]
        rowa = 15 * (a & (KEY_F - 1)) + 3 * (a >> 17) + 3
        rowb = 15 * (b & (KEY_F - 1)) + 3 * (b >> 17) + 3
        e2 = (k * 16 + iota) * 2
        cna = plsc.load_gather(conb, [e2])
        cnb_ = plsc.load_gather(conb, [e2 + 1])
        for comp in range(3):
            ixa[comp][sl] = jnp.where(a >= 0, rowa + comp, spread)
            ixb[comp][sl] = jnp.where(b >= 0, rowb + comp, spread)
            ixda[comp][sl] = 15 * cna + comp
            ixdb[comp][sl] = 15 * cnb_ + comp
        return 0
    lax.fori_loop(0, CHUNKS, idx_body, 0)

    copies = []
    for ix, dst in ((ixa0, ga0), (ixa1, ga1), (ixa2, ga2),
                    (ixb0, gb0), (ixb1, gb1), (ixb2, gb2),
                    (ixda0, gda0), (ixda1, gda1), (ixda2, gda2),
                    (ixdb0, gdb0), (ixdb1, gdb1), (ixdb2, gdb2)):
        copies.append(pltpu.async_copy(pred_hbm.at[ix], dst, sem))

    zero = jnp.zeros((16,), jnp.float32)
    nacc = [zero] * 11
    node_base = wid * SLICE
    for blk in range(SLICE // NB):
        nlo = node_base + blk * NB
        nb0 = jnp.minimum(nlo, N_NODES - NB)
        nstage = [
            pltpu.async_copy(pred_hbm.at[pl.ds(nb0 * 15, NB * 15)], pnb,
                             sem2),
            pltpu.async_copy(fm_hbm.at[pl.ds(nb0 * 4, NB * 4)], fmb, sem2),
            pltpu.async_copy(bcd_hbm.at[pl.ds(nb0, NB)], bcdb, sem2),
            pltpu.async_copy(bcr_hbm.at[pl.ds(nb0, NB)], bcrb, sem2),
            pltpu.async_copy(fe_hbm.at[pl.ds(nb0 * 3, NB * 3)], feb, sem2),
        ]
        for cp in nstage:
            cp.wait()

        def node_body(k, carry):
            (aq0, aq1, acf, aq2, aq3, acff, aq4, aq5, aq6, acsd, acsr) = carry
            sl = pl.ds(k * 16, 16)
            lane = k * 16 + iota
            nid = nb0 + lane
            m = ((nid >= nlo) & (nid < N_NODES)).astype(jnp.float32)
            p15 = lane * 15
            cols = [plsc.load_gather(pnb, [p15 + cc]) for cc in range(15)]
            bcd = bcdb[sl]
            bcr = bcrb[sl]
            free = m * (bcd < 0.5).astype(jnp.float32)
            supd = m * (bcd > 0.5).astype(jnp.float32)
            supr = m * (bcr > 0.5).astype(jnp.float32)
            f3 = lane * 3
            r0 = cols[3] + cols[6] + cols[9] + cols[12] \
                - plsc.load_gather(feb, [f3])
            r1 = cols[4] + cols[7] + cols[10] + cols[13] \
                - plsc.load_gather(feb, [f3 + 1])
            r2 = cols[5] + cols[8] + cols[11] + cols[14] \
                - plsc.load_gather(feb, [f3 + 2])
            aq0 = aq0 + free * (r0 * r0 + r1 * r1)
            aq1 = aq1 + free * (r2 * r2)
            acf = acf + free
            f4 = lane * 4
            for f in range(4):
                ffree = m * (plsc.load_gather(fmb, [f4 + f]) < 0.5).astype(
                    jnp.float32)
                g0 = cols[3 + 3 * f]
                g1 = cols[4 + 3 * f]
                g2 = cols[5 + 3 * f]
                aq2 = aq2 + ffree * (g0 * g0 + g1 * g1)
                aq3 = aq3 + ffree * (g2 * g2)
                acff = acff + ffree
            aq4 = aq4 + supd * cols[0] * cols[0]
            aq5 = aq5 + supd * cols[1] * cols[1]
            aq6 = aq6 + supr * cols[2] * cols[2]
            acsd = acsd + supd
            acsr = acsr + supr
            return (aq0, aq1, acf, aq2, aq3, acff, aq4, aq5, aq6, acsd, acsr)

        nacc = list(lax.fori_loop(0, NBCH, node_body, tuple(nacc)))

    for cp in copies:
        cp.wait()

    def phys_body(k, carry):
        acc_n, acc_m, acc_v = carry
        sl = pl.ds(k * 16, 16)
        e3 = (k * 16 + iota) * 3
        eidv = eb + k * 16 + iota
        oka = (mka[sl] >= 0).astype(jnp.float32)
        okb = (mkb[sl] >= 0).astype(jnp.float32)

        fa0 = ga0[sl] * oka
        fa1 = ga1[sl] * oka
        fa2 = ga2[sl] * oka
        fb0 = gb0[sl] * okb
        fb1 = gb1[sl] * okb
        fb2 = gb2[sl] * okb
        da0 = gda0[sl]
        da1 = gda1[sl]
        da2 = gda2[sl]
        db0 = gdb0[sl]
        db1 = gdb1[sl]
        db2 = gdb2[sl]

        cs = plsc.load_gather(dirb, [e3])
        sn = plsc.load_gather(dirb, [e3 + 2])
        lv = lb[sl]
        ea = peb[sl] * pab[sl]
        ei = peb[sl] * pib[sl]

        u_a = da0 * cs + da1 * sn
        w_a = -da0 * sn + da1 * cs
        t_a = da2
        u_b = db0 * cs + db1 * sn
        w_b = -db0 * sn + db1 * cs
        t_b = db2
        fra0 = fa0 * cs + fa1 * sn
        fra1 = -fa0 * sn + fa1 * cs
        fra2 = fa2
        frb0 = fb0 * cs + fb1 * sn
        frb1 = -fb0 * sn + fb1 * cs
        frb2 = fb2

        l2 = lv * lv
        l3 = l2 * lv
        n_sf = ea * (u_b - u_a) / lv
        m_a = ei / l2 * (-6.0 * w_a - 4.0 * lv * t_a + 6.0 * w_b - 2.0 * lv * t_b)
        m_b = ei / l2 * (6.0 * w_a + 2.0 * lv * t_a - 6.0 * w_b + 4.0 * lv * t_b)
        v_sf = ei / l3 * (12.0 * w_a + 6.0 * lv * t_a - 12.0 * w_b + 6.0 * lv * t_b)

        m = ((eidv >= lo) & (eidv < N_ELEMS)).astype(jnp.float32)
        rn0 = fra0 + n_sf
        rn1 = frb0 - n_sf
        rm0 = fra2 + m_a
        rm1 = frb2 - m_b
        rv0 = fra1 + v_sf
        rv1 = frb1 - v_sf
        acc_n = acc_n + m * (rn0 * rn0 + rn1 * rn1)
        acc_m = acc_m + m * (rm0 * rm0 + rm1 * rm1)
        acc_v = acc_v + m * (rv0 * rv0 + rv1 * rv1)
        return acc_n, acc_m, acc_v

    acc_n, acc_m, acc_v = lax.fori_loop(0, CHUNKS, phys_body,
                                        (zero, zero, zero))

    slots = [acc_n, acc_m, acc_v] + nacc
    for i, v in enumerate(slots):
        obuf[pl.ds(i * SLOTS, SLOTS)] = v
    obuf[pl.ds(14 * SLOTS, SLOTS)] = zero
    obuf[pl.ds(15 * SLOTS, SLOTS)] = zero
    pltpu.sync_copy(obuf, spmf.at[pl.ds(s * 256, 256)])
    plsc.subcore_barrier()

    @pl.when(s == 0)
    def _():
        pltpu.sync_copy(spmf, part_hbm.at[pl.ds(c * 16 * 256, 16 * 256)])


def _final_tc_body(ep_ref, fc_ref, mc_ref, uc_ref, tc_ref, out_ref):
    ep = ep_ref[...]
    fc = fc_ref[0, 0]
    mc = mc_ref[0, 0]
    uc = uc_ref[0, 0]
    th = tc_ref[0, 0]
    fc2 = fc * fc
    mc2 = mc * mc

    def slot(i):
        return jnp.sum(ep[:, 16 * i:16 * (i + 1)])

    l_eq = (slot(3) / fc2 + slot(4) / mc2) / jnp.maximum(slot(5), 1.0)
    l_free = (slot(6) / fc2 + slot(7) / mc2) / jnp.maximum(slot(8) * 3.0, 1.0)
    l_sup = ((slot(9) + slot(10)) / (uc * uc) / jnp.maximum(slot(12), 1.0)
             + slot(11) / (th * th) / jnp.maximum(slot(13), 1.0))
    e_cnt = float(N_ELEMS)
    total = (l_eq + l_free + l_sup
             + slot(0) / fc2 / e_cnt + slot(1) / mc2 / e_cnt
             + slot(2) / fc2 / e_cnt)
    out_ref[...] = jnp.reshape(total, (1, 1))


def _final_tc(ep, fc, mc, uc, th):
    return pl.pallas_call(
        _final_tc_body,
        in_specs=[
            pl.BlockSpec((NW, 256), lambda: (0, 0)),
            pl.BlockSpec((1, 1), lambda: (0, 0)),
            pl.BlockSpec((1, 1), lambda: (0, 0)),
            pl.BlockSpec((1, 1), lambda: (0, 0)),
            pl.BlockSpec((1, 1), lambda: (0, 0)),
        ],
        out_specs=pl.BlockSpec((1, 1), lambda: (0, 0)),
        out_shape=jax.ShapeDtypeStruct((1, 1), jnp.float32),
    )(ep, fc, mc, uc, th)


def kernel(pred, connectivity, face_element_id, face_is_A_end, face_mask,
           F_ext, bc_disp, bc_rot, elem_directions, elem_lengths,
           prop_E, prop_A, prop_I22, F_c, M_c, u_c, theta_c):
    pred_f = pred.reshape(-1)
    conn_f = connectivity.reshape(-1)
    dirs_f = elem_directions.reshape(-1)
    fm_f = face_mask.reshape(-1)
    bcd_f = bc_disp.reshape(-1)
    bcr_f = bc_rot.reshape(-1)
    fe_f = F_ext.reshape(-1)
    mka, mkb = _s1(face_element_id.reshape(-1),
                   face_is_A_end.reshape(-1), fm_f)
    part = _s2(mka, mkb, pred_f, conn_f, dirs_f, elem_lengths,
               prop_E, prop_A, prop_I22, fm_f, bcd_f, bcr_f, fe_f)
    out = _final_tc(part.reshape(NW, 256), F_c.reshape(1, 1),
                    M_c.reshape(1, 1), u_c.reshape(1, 1),
                    theta_c.reshape(1, 1))
    return out[0, 0]
